# Initial kernel scaffold; baseline (speedup 1.0000x reference)
import functools

import jax
import jax.numpy as jnp
from jax import lax
from jax.experimental import pallas as pl
from jax.experimental.pallas import tpu as pltpu
from jax.experimental.pallas import tpu_sc as plsc

D = 32
BATCH = 4096
SEQ = 200
NC = 2
NS = 16
NW = NC * NS
RW = BATCH // NW
K = SEQ // 2
CPW = RW * 2
TOT = BATCH * SEQ

_mesh = plsc.VectorSubcoreMesh(core_axis_name="c", subcore_axis_name="s")


@functools.partial(
    pl.kernel,
    out_type=(
        jax.ShapeDtypeStruct((TOT, D), jnp.float32),
        jax.ShapeDtypeStruct((BATCH, D), jnp.float32),
    ),
    mesh=_mesh,
    scratch_types=[
        pltpu.VMEM((CPW, K), jnp.int32),
        pltpu.VMEM((K, D), jnp.float32),
        pltpu.VMEM((RW, D), jnp.float32),
        pltpu.SemaphoreType.DMA,
    ],
)
def _embed_pool(ids_hbm, table_hbm, x_hbm, mean_hbm, idx_v, rows_v, mean_v, sem):
    wid = lax.axis_index("s") * NC + lax.axis_index("c")
    pltpu.sync_copy(ids_hbm.at[pl.ds(wid * CPW, CPW)], idx_v)
    inv = jnp.float32(1.0 / SEQ)
    zero = jnp.zeros((16,), jnp.float32)

    def chunk_body(c, carry):
        a0, a1 = carry
        pltpu.async_copy(table_hbm.at[idx_v.at[c]], rows_v, sem).wait()

        def row_body(j, acc):
            b0, b1 = acc
            return (b0 + rows_v[j, pl.ds(0, 16)], b1 + rows_v[j, pl.ds(16, 16)])

        a0, a1 = lax.fori_loop(0, K, row_body, (a0, a1))
        pltpu.sync_copy(rows_v, x_hbm.at[pl.ds(wid * (CPW * K) + c * K, K)])

        r = c // 2
        odd = c % 2 == 1

        @pl.when(odd)
        def _():
            mean_v[r, pl.ds(0, 16)] = a0 * inv
            mean_v[r, pl.ds(16, 16)] = a1 * inv

        a0 = jnp.where(odd, zero, a0)
        a1 = jnp.where(odd, zero, a1)
        return (a0, a1)

    lax.fori_loop(0, CPW, chunk_body, (zero, zero))
    pltpu.sync_copy(mean_v, mean_hbm.at[pl.ds(wid * RW, RW)])


def kernel(input_ids, embedding_weight):
    ids2d = input_ids.reshape(TOT // K, K)
    x_flat, mean = _embed_pool(ids2d, embedding_weight)
    return x_flat.reshape(BATCH, SEQ, D), mean


# speedup vs baseline: 1.2919x; 1.2919x over previous
import functools

import jax
import jax.numpy as jnp
from jax import lax
from jax.experimental import pallas as pl
from jax.experimental.pallas import tpu as pltpu
from jax.experimental.pallas import tpu_sc as plsc

D = 32
BATCH = 4096
SEQ = 200
NC = 2
NS = 16
NW = NC * NS
RW = BATCH // NW
K = SEQ // 2
CPW = RW * 2
TOT = BATCH * SEQ

_mesh = plsc.VectorSubcoreMesh(core_axis_name="c", subcore_axis_name="s")


@functools.partial(
    pl.kernel,
    out_type=(
        jax.ShapeDtypeStruct((TOT, D), jnp.float32),
        jax.ShapeDtypeStruct((BATCH, D), jnp.float32),
    ),
    mesh=_mesh,
    compiler_params=pltpu.CompilerParams(use_tc_tiling_on_sc=False),
    scratch_types=[
        pltpu.VMEM((CPW, K), jnp.int32),
        pltpu.VMEM((SEQ, D), jnp.float32),
        pltpu.VMEM((RW, D), jnp.float32),
        pltpu.SemaphoreType.DMA,
    ],
)
def _embed_pool(ids_hbm, table_hbm, x_hbm, mean_hbm, idx_v, rows_v, mean_v, sem):
    wid = lax.axis_index("s") * NC + lax.axis_index("c")
    pltpu.sync_copy(ids_hbm.at[pl.ds(wid * CPW, CPW)], idx_v)
    inv = jnp.float32(1.0 / SEQ)
    zero = jnp.zeros((16,), jnp.float32)

    def row_chunk(r, _):
        cp0 = pltpu.async_copy(
            table_hbm.at[idx_v.at[2 * r]], rows_v.at[pl.ds(0, K)], sem)
        cp1 = pltpu.async_copy(
            table_hbm.at[idx_v.at[2 * r + 1]], rows_v.at[pl.ds(K, K)], sem)
        cp0.wait()
        cp1.wait()

        def acc_body(j, acc):
            b0, b1 = acc
            return (b0 + rows_v[j, pl.ds(0, 16)], b1 + rows_v[j, pl.ds(16, 16)])

        a0, a1 = lax.fori_loop(0, SEQ, acc_body, (zero, zero))
        mean_v[r, pl.ds(0, 16)] = a0 * inv
        mean_v[r, pl.ds(16, 16)] = a1 * inv
        pltpu.sync_copy(rows_v, x_hbm.at[pl.ds(wid * (RW * SEQ) + r * SEQ, SEQ)])
        return 0

    lax.fori_loop(0, RW, row_chunk, 0)
    pltpu.sync_copy(mean_v, mean_hbm.at[pl.ds(wid * RW, RW)])


def kernel(input_ids, embedding_weight):
    ids2d = input_ids.reshape(TOT // K, K)
    x_flat, mean = _embed_pool(ids2d, embedding_weight)
    return x_flat.reshape(BATCH, SEQ, D), mean


# speedup vs baseline: 1.5286x; 1.1832x over previous
import functools

import jax
import jax.numpy as jnp
from jax import lax
from jax.experimental import pallas as pl
from jax.experimental.pallas import tpu as pltpu
from jax.experimental.pallas import tpu_sc as plsc

D = 32
BATCH = 4096
SEQ = 200
NC = 2
NS = 16
NW = NC * NS
RW = BATCH // NW
K = SEQ // 2
CPW = RW * 2
TOT = BATCH * SEQ
NB = 4
UNROLL = 8

_mesh = plsc.VectorSubcoreMesh(core_axis_name="c", subcore_axis_name="s")


@functools.partial(
    pl.kernel,
    out_type=(
        jax.ShapeDtypeStruct((TOT, D), jnp.float32),
        jax.ShapeDtypeStruct((BATCH, D), jnp.float32),
    ),
    mesh=_mesh,
    compiler_params=pltpu.CompilerParams(use_tc_tiling_on_sc=False),
    scratch_types=[
        pltpu.VMEM((CPW, K), jnp.int32),
        pltpu.VMEM((NB, SEQ, D), jnp.float32),
        pltpu.VMEM((RW, D), jnp.float32),
        pltpu.SemaphoreType.DMA,
        pltpu.SemaphoreType.DMA,
        pltpu.SemaphoreType.DMA,
        pltpu.SemaphoreType.DMA,
        pltpu.SemaphoreType.DMA,
        pltpu.SemaphoreType.DMA,
        pltpu.SemaphoreType.DMA,
        pltpu.SemaphoreType.DMA,
    ],
)
def _embed_pool(ids_hbm, table_hbm, x_hbm, mean_hbm, idx_v, rows_v, mean_v,
                g0, g1, g2, g3, w0, w1, w2, w3):
    gsems = (g0, g1, g2, g3)
    wsems = (w0, w1, w2, w3)
    wid = lax.axis_index("s") * NC + lax.axis_index("c")
    pltpu.sync_copy(ids_hbm.at[pl.ds(wid * CPW, CPW)], idx_v)
    inv = jnp.float32(1.0 / SEQ)
    zero = jnp.zeros((16,), jnp.float32)
    x_base = wid * (RW * SEQ)

    def start_gather(r, b):
        buf = rows_v.at[b]
        pltpu.async_copy(table_hbm.at[idx_v.at[2 * r]],
                         buf.at[pl.ds(0, K)], gsems[b])
        pltpu.async_copy(table_hbm.at[idx_v.at[2 * r + 1]],
                         buf.at[pl.ds(K, K)], gsems[b])

    for b in range(NB):
        start_gather(b, b)

    @pl.loop(0, RW, step=NB)
    def _group(g):
        for b in range(NB):
            r = g + b
            buf = rows_v.at[b]
            pltpu.make_async_copy(x_hbm.at[pl.ds(0, SEQ)], buf, gsems[b]).wait()
            wcp = pltpu.async_copy(
                buf, x_hbm.at[pl.ds(x_base + r * SEQ, SEQ)], wsems[b])

            def acc_step(t, accs, buf=buf):
                accs = list(accs)
                for k in range(UNROLL):
                    j = t * UNROLL + k
                    c = k % 4
                    accs[2 * c] = accs[2 * c] + buf[j, pl.ds(0, 16)]
                    accs[2 * c + 1] = accs[2 * c + 1] + buf[j, pl.ds(16, 16)]
                return tuple(accs)

            accs = lax.fori_loop(0, SEQ // UNROLL, acc_step, (zero,) * 8)
            a0 = (accs[0] + accs[2]) + (accs[4] + accs[6])
            a1 = (accs[1] + accs[3]) + (accs[5] + accs[7])
            mean_v[r, pl.ds(0, 16)] = a0 * inv
            mean_v[r, pl.ds(16, 16)] = a1 * inv

            wcp.wait()

            @pl.when(r + NB < RW)
            def _():
                start_gather(r + NB, b)

    pltpu.sync_copy(mean_v, mean_hbm.at[pl.ds(wid * RW, RW)])


def kernel(input_ids, embedding_weight):
    ids2d = input_ids.reshape(TOT // K, K)
    x_flat, mean = _embed_pool(ids2d, embedding_weight)
    return x_flat.reshape(BATCH, SEQ, D), mean
